# baseline (speedup 1.0000x reference)
import jax
import jax.numpy as jnp
from jax.experimental import pallas as pl

ATOM_FDIM = 128
BOND_FDIM = 6
H = 64
KHEAD = 2
DEPTH = 2
N = 64
NBS = 8
BM = 32

_F32 = jnp.float32


def _lrelu(x, s):
    return jnp.maximum(x, s * x)


def _bdot(a, b):
    return jax.lax.dot_general(
        a, b, (((2,), (1,)), ((0,), (0,))), preferred_element_type=_F32)


def _prep_params(params):
    out = []

    def lin(p):
        W, b = p
        out.append(W.T)
        out.append(b.reshape(1, -1))

    lin(params["vertex_embedding"])
    for it in range(DEPTH):
        U2, U2b = params["label_U2"][it]
        for k in range(KHEAD):
            lin(params["W_a_main"][it][k])
            lin(params["W_main"][it][k])
        out.append(U2[:, :H].T)
        sf_w = jnp.concatenate(
            [params["W_a_super"][it][0][0].T, params["W_a_super"][it][1][0].T,
             params["W_super_to_main"][it][0].T, params["W_super"][it][0].T],
            axis=1)
        sf_b = jnp.concatenate(
            [params["W_a_super"][it][0][1], params["W_a_super"][it][1][1],
             params["W_super_to_main"][it][1], params["W_super"][it][1]]
        ).reshape(1, 4 * H)
        out.append(sf_w)
        out.append(sf_b)
        out.append(jnp.concatenate(
            [params["W_bmm"][it][0][0].reshape(1, H),
             params["W_bmm"][it][1][0].reshape(1, H)], axis=1))
        out.append(jnp.concatenate(
            [jnp.broadcast_to(params["W_bmm"][it][0][1].reshape(1, 1) / H, (1, H)),
             jnp.broadcast_to(params["W_bmm"][it][1][1].reshape(1, 1) / H, (1, H))],
            axis=1))
        out.append(U2[:, H:].T)
        out.append(U2b.reshape(1, -1) / 2)
        U1, U1b = params["label_U1"][it]
        out.append(U1[:, :H].T)
        out.append(U1[:, H:].T)
        out.append(U1b.reshape(1, -1))
        m2sWf, m2sbf = params["W_main_to_super"][it]
        out.append(m2sWf[:, :H].T)
        out.append(m2sWf[:, H:].T)
        out.append(m2sbf.reshape(1, -1))
        lin(params["W_zm1"][it])
        lin(params["W_zm2"][it])
        lin(params["W_zs1"][it])
        lin(params["W_zs2"][it])
    return out


def _body(*refs):
    (af_ref, bf_ref, anb_ref, bnb_ref, nmask_ref, dmask_ref, pat_ref) = refs[:7]
    vf_out, sf_out = refs[-2:]
    prefs = list(refs[7:-2])

    def nxt():
        return prefs.pop(0)[...]

    veW, veb = nxt(), nxt()
    depth_p = [tuple(nxt() for _ in range(29)) for _ in range(DEPTH)]

    AF = af_ref[...].reshape(BM * N, ATOM_FDIM)
    bf = bf_ref[...].reshape(BM * N, BOND_FDIM)
    anb = anb_ref[...]
    bnb = bnb_ref[...]
    nmask_row = nmask_ref[...]
    dmask3 = dmask_ref[...]
    dmask_row = dmask3.reshape(BM, N)

    vf = _lrelu(jnp.dot(AF, veW, preferred_element_type=_F32) + veb, 0.01)
    sf = _bdot(dmask3, vf.reshape(BM, N, H)).reshape(BM, H)

    iota2 = jax.lax.broadcasted_iota(jnp.int16, (BM, N * NBS, 2 * N), 2)
    target = jnp.where(iota2 < N, anb, bnb)
    oh = (target == (iota2 & (N - 1))).astype(jnp.bfloat16)

    R = (pat_ref[...] * nmask_row).astype(jnp.bfloat16)

    for it in range(DEPTH):
        (am0W, am0b, m0W, m0b, am1W, am1b, m1W, m1b, U2v, sfW, sfb,
         bmm_w, bmm_bd, U2e, U2b, U1v, U1n, U1b,
         m2sW0, m2sW1, m2sb, zm1W, zm1b, zm2W, zm2b, zs1W, zs1b, zs2W, zs2b
         ) = depth_p[it]
        T = jnp.tanh(jnp.dot(sf, sfW, preferred_element_type=_F32) + sfb)
        Cpair = (T[:, :2 * H] * bmm_w).reshape(BM, 2 * H)
        s2m = T[:, 2 * H:3 * H]
        ss = T[:, 3 * H:]
        heads = []
        for k in range(KHEAD):
            amW, amb = (am0W, am0b) if k == 0 else (am1W, am1b)
            mW, mb = (m0W, m0b) if k == 0 else (m1W, m1b)
            AM = jnp.tanh(jnp.dot(vf, amW, preferred_element_type=_F32) + amb)
            V = jnp.dot(vf, mW, preferred_element_type=_F32) + mb
            C = Cpair[:, k * H:(k + 1) * H].reshape(BM, 1, H)
            t = (AM.reshape(BM, N, H) * C
                 + bmm_bd[:, k * H:(k + 1) * H].reshape(1, 1, H))
            a = jnp.sum(t, axis=-1)
            aexp = jnp.exp(a) * dmask_row
            den = jnp.sum(aexp, axis=-1, keepdims=True).reshape(BM, 1, 1)
            pooled = _bdot(aexp.reshape(BM, 1, N), V.reshape(BM, N, H))
            heads.append(pooled / (den + 1e-6))
        h0 = heads[0].reshape(BM, H)
        h1 = heads[1].reshape(BM, H)
        mts = jnp.tanh(jnp.dot(h0, m2sW0, preferred_element_type=_F32)
                       + jnp.dot(h1, m2sW1, preferred_element_type=_F32) + m2sb)

        Pv = jnp.dot(vf, U2v, preferred_element_type=_F32) + U2b
        Pe = jnp.dot(bf, U2e, preferred_element_type=_F32) + U2b
        Pcat = jnp.concatenate(
            [Pv.reshape(BM, N, H), Pe.reshape(BM, N, H)], axis=1)
        G = _bdot(oh, Pcat)
        L = _lrelu(G, 0.1)
        nei = _bdot(R, L).reshape(BM * N, H)
        main_self = _lrelu(
            jnp.dot(vf, U1v, preferred_element_type=_F32)
            + jnp.dot(nei, U1n, preferred_element_type=_F32) + U1b, 0.1)

        zm_in = (jnp.dot(main_self, zm1W, preferred_element_type=_F32) + zm1b
                 ).reshape(BM, N, H)
        zm_sup = jnp.dot(s2m, zm2W, preferred_element_type=_F32) + zm2b
        zm = jax.nn.sigmoid(zm_in + zm_sup.reshape(BM, 1, H))
        vf = ((1 - zm) * main_self.reshape(BM, N, H)
              + zm * s2m.reshape(BM, 1, H)).reshape(BM * N, H)
        zs = jax.nn.sigmoid(
            jnp.dot(ss, zs1W, preferred_element_type=_F32) + zs1b
            + jnp.dot(mts, zs2W, preferred_element_type=_F32) + zs2b)
        sf = (1 - zs) * ss + zs * mts

    vf_out[...] = vf.reshape(BM, N, H)
    sf_out[...] = sf.reshape(BM, 1, H)


def kernel(batch_size, atom_fea, bond_fea, d_anb, d_bnb, d_nbs_mask, d_mask, params):
    B = atom_fea.shape[0]
    flat = _prep_params(params)

    anb2 = d_anb.astype(jnp.int16).reshape(B, N * NBS, 1)
    bnb2 = d_bnb.astype(jnp.int16).reshape(B, N * NBS, 1)
    nmask2 = d_nbs_mask.reshape(B, 1, N * NBS)
    dmask2 = d_mask.reshape(B, 1, N)
    pat = jnp.repeat(jnp.eye(N, dtype=jnp.float32), NBS, axis=1
                     ).reshape(1, N, N * NBS)

    def rep(shape):
        nd = len(shape)
        return pl.BlockSpec(shape, lambda i, _n=nd: (0,) * _n)

    in_specs = [
        pl.BlockSpec((BM, N, ATOM_FDIM), lambda i: (i, 0, 0)),
        pl.BlockSpec((BM, N, BOND_FDIM), lambda i: (i, 0, 0)),
        pl.BlockSpec((BM, N * NBS, 1), lambda i: (i, 0, 0)),
        pl.BlockSpec((BM, N * NBS, 1), lambda i: (i, 0, 0)),
        pl.BlockSpec((BM, 1, N * NBS), lambda i: (i, 0, 0)),
        pl.BlockSpec((BM, 1, N), lambda i: (i, 0, 0)),
        rep((1, N, N * NBS)),
    ] + [rep(a.shape) for a in flat]

    out_shape = (
        jax.ShapeDtypeStruct((B, N, H), jnp.float32),
        jax.ShapeDtypeStruct((B, 1, H), jnp.float32),
    )
    out_specs = (
        pl.BlockSpec((BM, N, H), lambda i: (i, 0, 0)),
        pl.BlockSpec((BM, 1, H), lambda i: (i, 0, 0)),
    )
    vf, sf = pl.pallas_call(
        _body,
        grid=(B // BM,),
        in_specs=in_specs,
        out_specs=out_specs,
        out_shape=out_shape,
    )(atom_fea, bond_fea, anb2, bnb2, nmask2, dmask2, pat, *flat)
    return vf, sf


# speedup vs baseline: 1.0157x; 1.0157x over previous
import jax
import jax.numpy as jnp
import numpy as np
from jax.experimental import pallas as pl

ATOM_FDIM = 128
BOND_FDIM = 6
H = 64
KHEAD = 2
DEPTH = 2
N = 64
NBS = 8
BM = 32

_F32 = jnp.float32

_PAT = np.repeat(np.eye(N, dtype=np.float32), NBS, axis=1).reshape(1, N, N * NBS)


def _lrelu(x, s):
    return jnp.maximum(x, s * x)


def _bdot(a, b):
    return jax.lax.dot_general(
        a, b, (((2,), (1,)), ((0,), (0,))), preferred_element_type=_F32)


def _dot(x, w, b=None):
    r = jnp.dot(x, w, preferred_element_type=_F32)
    return r if b is None else r + b


_NSQ = 12
_NB = 15


def _prep_params(params):
    sq, wide, u2s, biases, bmmw, bmmbd = [], [], [], [], [], []
    biases.append(params["vertex_embedding"][1])
    for it in range(DEPTH):
        sq.extend([
            params["W_a_main"][it][0][0], params["W_main"][it][0][0],
            params["W_a_main"][it][1][0], params["W_main"][it][1][0],
            params["W_a_super"][it][0][0], params["W_a_super"][it][1][0],
            params["W_super_to_main"][it][0], params["W_super"][it][0],
            params["W_zm1"][it][0], params["W_zm2"][it][0],
            params["W_zs1"][it][0], params["W_zs2"][it][0],
        ])
        wide.extend([params["label_U1"][it][0], params["W_main_to_super"][it][0]])
        u2s.append(params["label_U2"][it][0])
        biases.extend([
            params["W_a_main"][it][0][1], params["W_main"][it][0][1],
            params["W_a_main"][it][1][1], params["W_main"][it][1][1],
            params["W_a_super"][it][0][1], params["W_a_super"][it][1][1],
            params["W_super_to_main"][it][1], params["W_super"][it][1],
            params["label_U2"][it][1] * 0.5,
            params["label_U1"][it][1], params["W_main_to_super"][it][1],
            params["W_zm1"][it][1], params["W_zm2"][it][1],
            params["W_zs1"][it][1], params["W_zs2"][it][1],
        ])
        for k in range(KHEAD):
            bmmw.append(params["W_bmm"][it][k][0].reshape(H))
            bmmbd.append(jnp.broadcast_to(params["W_bmm"][it][k][1] / H, (H,)))
    return [
        params["vertex_embedding"][0].T,
        jnp.stack(sq).transpose(0, 2, 1),
        jnp.stack(wide).transpose(0, 2, 1),
        jnp.stack(u2s).transpose(0, 2, 1),
        jnp.stack(biases),
        jnp.stack(bmmw),
        jnp.stack(bmmbd),
    ]


def _body(af_ref, bf_ref, anb_ref, bnb_ref, nmask_ref, dmask_ref, pat_ref,
          vew_ref, sq_ref, wide_ref, u2_ref, bias_ref, bmmw_ref, bmmbd_ref,
          vf_out, sf_out):
    SQ = sq_ref[...]
    WIDE = wide_ref[...]
    U2T = u2_ref[...]
    BIAS = bias_ref[...]

    AF = af_ref[...].reshape(BM * N, ATOM_FDIM)
    bf = bf_ref[...].reshape(BM * N, BOND_FDIM)
    anb = anb_ref[...]
    bnb = bnb_ref[...]
    nmask_row = nmask_ref[...]
    dmask3 = dmask_ref[...]
    dmask_row = dmask3.reshape(BM, N)

    vf = _lrelu(_dot(AF, vew_ref[...], BIAS[0:1]), 0.01)
    sf = _bdot(dmask3, vf.reshape(BM, N, H)).reshape(BM, H)

    iota2 = jax.lax.broadcasted_iota(jnp.int16, (BM, N * NBS, 2 * N), 2)
    target = jnp.where(iota2 < N, anb, bnb)
    oh = (target == (iota2 & (N - 1))).astype(jnp.bfloat16)

    R = (pat_ref[...] * nmask_row).astype(jnp.bfloat16)

    for it in range(DEPTH):
        o, ob, ow = it * _NSQ, 1 + it * _NB, 2 * it
        sfW = jnp.concatenate([SQ[o + 4], SQ[o + 5], SQ[o + 6], SQ[o + 7]],
                              axis=1)
        sfb = jnp.concatenate(
            [BIAS[ob + 4:ob + 5], BIAS[ob + 5:ob + 6], BIAS[ob + 6:ob + 7],
             BIAS[ob + 7:ob + 8]], axis=1)
        T = jnp.tanh(_dot(sf, sfW, sfb))
        BMW = bmmw_ref[...]
        bmmw2 = jnp.concatenate([BMW[2 * it:2 * it + 1],
                                 BMW[2 * it + 1:2 * it + 2]], axis=1)
        Cpair = (T[:, :2 * H] * bmmw2).reshape(BM, 2 * H)
        s2m = T[:, 2 * H:3 * H]
        ss = T[:, 3 * H:]
        heads = []
        for k in range(KHEAD):
            AM = jnp.tanh(_dot(vf, SQ[o + 2 * k], BIAS[ob + 2 * k:ob + 2 * k + 1]))
            V = _dot(vf, SQ[o + 2 * k + 1], BIAS[ob + 2 * k + 1:ob + 2 * k + 2])
            C = Cpair[:, k * H:(k + 1) * H].reshape(BM, 1, H)
            r = 2 * it + k
            t = (AM.reshape(BM, N, H) * C
                 + bmmbd_ref[r:r + 1, :].reshape(1, 1, H))
            a = jnp.sum(t, axis=-1)
            aexp = jnp.exp(a) * dmask_row
            den = jnp.sum(aexp, axis=-1, keepdims=True).reshape(BM, 1, 1)
            pooled = _bdot(aexp.reshape(BM, 1, N), V.reshape(BM, N, H))
            heads.append(pooled / (den + 1e-6))
        h0 = heads[0].reshape(BM, H)
        h1 = heads[1].reshape(BM, H)
        m2sT = WIDE[ow + 1]
        mts = jnp.tanh(_dot(h0, m2sT[:H]) + _dot(h1, m2sT[H:])
                       + BIAS[ob + 10:ob + 11])

        u2b2 = BIAS[ob + 8:ob + 9]
        Pv = _dot(vf, U2T[it][:H], u2b2)
        Pe = _dot(bf, U2T[it][H:], u2b2)
        Pcat = jnp.concatenate(
            [Pv.reshape(BM, N, H), Pe.reshape(BM, N, H)], axis=1)
        G = _bdot(oh, Pcat)
        L = _lrelu(G, 0.1)
        nei = _bdot(R, L).reshape(BM * N, H)
        U1T = WIDE[ow]
        main_self = _lrelu(
            _dot(vf, U1T[:H]) + _dot(nei, U1T[H:]) + BIAS[ob + 9:ob + 10], 0.1)

        zm_in = _dot(main_self, SQ[o + 8], BIAS[ob + 11:ob + 12]
                     ).reshape(BM, N, H)
        zm_sup = _dot(s2m, SQ[o + 9], BIAS[ob + 12:ob + 13])
        zm = jax.nn.sigmoid(zm_in + zm_sup.reshape(BM, 1, H))
        vf = ((1 - zm) * main_self.reshape(BM, N, H)
              + zm * s2m.reshape(BM, 1, H)).reshape(BM * N, H)
        zs = jax.nn.sigmoid(_dot(ss, SQ[o + 10], BIAS[ob + 13:ob + 14])
                            + _dot(mts, SQ[o + 11], BIAS[ob + 14:ob + 15]))
        sf = (1 - zs) * ss + zs * mts

    vf_out[...] = vf.reshape(BM, N, H)
    sf_out[...] = sf.reshape(BM, 1, H)


def kernel(batch_size, atom_fea, bond_fea, d_anb, d_bnb, d_nbs_mask, d_mask, params):
    B = atom_fea.shape[0]
    flat = _prep_params(params)

    anb2 = d_anb.astype(jnp.int16).reshape(B, N * NBS, 1)
    bnb2 = d_bnb.astype(jnp.int16).reshape(B, N * NBS, 1)
    nmask2 = d_nbs_mask.reshape(B, 1, N * NBS)
    dmask2 = d_mask.reshape(B, 1, N)

    def rep(shape):
        nd = len(shape)
        return pl.BlockSpec(shape, lambda i, _n=nd: (0,) * _n)

    in_specs = [
        pl.BlockSpec((BM, N, ATOM_FDIM), lambda i: (i, 0, 0)),
        pl.BlockSpec((BM, N, BOND_FDIM), lambda i: (i, 0, 0)),
        pl.BlockSpec((BM, N * NBS, 1), lambda i: (i, 0, 0)),
        pl.BlockSpec((BM, N * NBS, 1), lambda i: (i, 0, 0)),
        pl.BlockSpec((BM, 1, N * NBS), lambda i: (i, 0, 0)),
        pl.BlockSpec((BM, 1, N), lambda i: (i, 0, 0)),
        rep((1, N, N * NBS)),
    ] + [rep(a.shape) for a in flat]

    out_shape = (
        jax.ShapeDtypeStruct((B, N, H), jnp.float32),
        jax.ShapeDtypeStruct((B, 1, H), jnp.float32),
    )
    out_specs = (
        pl.BlockSpec((BM, N, H), lambda i: (i, 0, 0)),
        pl.BlockSpec((BM, 1, H), lambda i: (i, 0, 0)),
    )
    vf, sf = pl.pallas_call(
        _body,
        grid=(B // BM,),
        in_specs=in_specs,
        out_specs=out_specs,
        out_shape=out_shape,
    )(atom_fea, bond_fea, anb2, bnb2, nmask2, dmask2,
      jnp.asarray(_PAT), *flat)
    return vf, sf


# speedup vs baseline: 1.6137x; 1.5887x over previous
import jax
import jax.numpy as jnp
import numpy as np
from jax.experimental import pallas as pl

ATOM_FDIM = 128
BOND_FDIM = 6
H = 64
KHEAD = 2
DEPTH = 2
N = 64
NBS = 8
BM = 32

_F32 = jnp.float32

_PAT = np.repeat(np.eye(N, dtype=np.float32), NBS, axis=1).reshape(1, N, N * NBS)


def _lrelu(x, s):
    return jnp.maximum(x, s * x)


def _bdot(a, b):
    return jax.lax.dot_general(
        a, b, (((2,), (1,)), ((0,), (0,))), preferred_element_type=_F32)


def _dot(x, w, b=None):
    r = jnp.dot(x, w, preferred_element_type=_F32)
    return r if b is None else r + b


_NSQ = 12
_NB = 15


def _prep_params(params):
    sq, wide, u2s, biases, bmmw, bmmbd = [], [], [], [], [], []
    biases.append(params["vertex_embedding"][1])
    for it in range(DEPTH):
        sq.extend([
            params["W_a_main"][it][0][0], params["W_main"][it][0][0],
            params["W_a_main"][it][1][0], params["W_main"][it][1][0],
            params["W_a_super"][it][0][0], params["W_a_super"][it][1][0],
            params["W_super_to_main"][it][0], params["W_super"][it][0],
            params["W_zm1"][it][0], params["W_zm2"][it][0],
            params["W_zs1"][it][0], params["W_zs2"][it][0],
        ])
        wide.extend([params["label_U1"][it][0], params["W_main_to_super"][it][0]])
        u2s.append(params["label_U2"][it][0])
        biases.extend([
            params["W_a_main"][it][0][1], params["W_main"][it][0][1],
            params["W_a_main"][it][1][1], params["W_main"][it][1][1],
            params["W_a_super"][it][0][1], params["W_a_super"][it][1][1],
            params["W_super_to_main"][it][1], params["W_super"][it][1],
            params["label_U2"][it][1] * 0.5,
            params["label_U1"][it][1], params["W_main_to_super"][it][1],
            params["W_zm1"][it][1], params["W_zm2"][it][1],
            params["W_zs1"][it][1], params["W_zs2"][it][1],
        ])
        for k in range(KHEAD):
            bmmw.append(params["W_bmm"][it][k][0].reshape(H))
            bmmbd.append(jnp.broadcast_to(params["W_bmm"][it][k][1] / H, (H,)))
    return [
        params["vertex_embedding"][0].T,
        jnp.stack(sq).transpose(0, 2, 1),
        jnp.stack(wide).transpose(0, 2, 1),
        jnp.stack(u2s).transpose(0, 2, 1),
        jnp.stack(biases),
        jnp.stack(bmmw),
        jnp.stack(bmmbd),
    ]


def _body(af_ref, bf_ref, anb_ref, bnb_ref, nmask_ref, dmask_ref, pat_ref,
          vew_ref, sq_ref, wide_ref, u2_ref, bias_ref, bmmw_ref, bmmbd_ref,
          vf_out, sf_out):
    SQ = sq_ref[...]
    WIDE = wide_ref[...]
    U2T = u2_ref[...]
    BIAS = bias_ref[...]

    AF = af_ref[...].reshape(BM * N, ATOM_FDIM)
    bf = bf_ref[...].reshape(BM * N, BOND_FDIM)
    anb3 = anb_ref[...].reshape(BM, 1, N * NBS)
    bnb3 = bnb_ref[...].reshape(BM, 1, N * NBS)
    nmask_row = nmask_ref[...].reshape(BM, 1, N * NBS)
    dmask_row = dmask_ref[...]
    dmask3 = dmask_row.reshape(BM, 1, N)

    vf = _lrelu(_dot(AF, vew_ref[...], BIAS[0:1]), 0.01)
    sf = _bdot(dmask3, vf.reshape(BM, N, H)).reshape(BM, H)

    iotaT = jax.lax.broadcasted_iota(jnp.int16, (BM, 2 * N, N * NBS), 1)
    targetT = jnp.where(iotaT < N, anb3, bnb3)
    ohT = (targetT == (iotaT & (N - 1))).astype(jnp.bfloat16)

    R = (pat_ref[...] * nmask_row).astype(jnp.bfloat16)

    for it in range(DEPTH):
        o, ob, ow = it * _NSQ, 1 + it * _NB, 2 * it
        sfW = jnp.concatenate([SQ[o + 4], SQ[o + 5], SQ[o + 6], SQ[o + 7]],
                              axis=1)
        sfb = jnp.concatenate(
            [BIAS[ob + 4:ob + 5], BIAS[ob + 5:ob + 6], BIAS[ob + 6:ob + 7],
             BIAS[ob + 7:ob + 8]], axis=1)
        T = jnp.tanh(_dot(sf, sfW, sfb))
        BMW = bmmw_ref[...]
        bmmw2 = jnp.concatenate([BMW[2 * it:2 * it + 1],
                                 BMW[2 * it + 1:2 * it + 2]], axis=1)
        Cpair = (T[:, :2 * H] * bmmw2).reshape(BM, 2 * H)
        s2m = T[:, 2 * H:3 * H]
        ss = T[:, 3 * H:]
        heads = []
        for k in range(KHEAD):
            AM = jnp.tanh(_dot(vf, SQ[o + 2 * k], BIAS[ob + 2 * k:ob + 2 * k + 1]))
            V = _dot(vf, SQ[o + 2 * k + 1], BIAS[ob + 2 * k + 1:ob + 2 * k + 2])
            C = Cpair[:, k * H:(k + 1) * H].reshape(BM, 1, H)
            r = 2 * it + k
            t = (AM.reshape(BM, N, H) * C
                 + bmmbd_ref[r:r + 1, :].reshape(1, 1, H))
            a = jnp.sum(t, axis=-1)
            aexp = jnp.exp(a) * dmask_row
            den = jnp.sum(aexp, axis=-1, keepdims=True).reshape(BM, 1, 1)
            pooled = _bdot(aexp.reshape(BM, 1, N), V.reshape(BM, N, H))
            heads.append(pooled / (den + 1e-6))
        h0 = heads[0].reshape(BM, H)
        h1 = heads[1].reshape(BM, H)
        m2sT = WIDE[ow + 1]
        mts = jnp.tanh(_dot(h0, m2sT[:H]) + _dot(h1, m2sT[H:])
                       + BIAS[ob + 10:ob + 11])

        u2b2 = BIAS[ob + 8:ob + 9]
        Pv = _dot(vf, U2T[it][:H], u2b2)
        Pe = _dot(bf, U2T[it][H:], u2b2)
        Pcat = jnp.concatenate(
            [Pv.reshape(BM, N, H), Pe.reshape(BM, N, H)], axis=1)
        G = jax.lax.dot_general(
            ohT, Pcat, (((1,), (1,)), ((0,), (0,))),
            preferred_element_type=_F32)
        L = _lrelu(G, 0.1)
        nei = _bdot(R, L).reshape(BM * N, H)
        U1T = WIDE[ow]
        main_self = _lrelu(
            _dot(vf, U1T[:H]) + _dot(nei, U1T[H:]) + BIAS[ob + 9:ob + 10], 0.1)

        zm_in = _dot(main_self, SQ[o + 8], BIAS[ob + 11:ob + 12]
                     ).reshape(BM, N, H)
        zm_sup = _dot(s2m, SQ[o + 9], BIAS[ob + 12:ob + 13])
        zm = jax.nn.sigmoid(zm_in + zm_sup.reshape(BM, 1, H))
        vf = ((1 - zm) * main_self.reshape(BM, N, H)
              + zm * s2m.reshape(BM, 1, H)).reshape(BM * N, H)
        zs = jax.nn.sigmoid(_dot(ss, SQ[o + 10], BIAS[ob + 13:ob + 14])
                            + _dot(mts, SQ[o + 11], BIAS[ob + 14:ob + 15]))
        sf = (1 - zs) * ss + zs * mts

    vf_out[...] = vf.reshape(BM, N, H)
    sf_out[...] = sf.reshape(BM, 1, H)


def kernel(batch_size, atom_fea, bond_fea, d_anb, d_bnb, d_nbs_mask, d_mask, params):
    B = atom_fea.shape[0]
    flat = _prep_params(params)

    anb2 = d_anb.reshape(B, N * NBS).astype(jnp.int16)
    bnb2 = d_bnb.reshape(B, N * NBS).astype(jnp.int16)
    nmask2 = d_nbs_mask.reshape(B, N * NBS)
    dmask2 = d_mask

    def rep(shape):
        nd = len(shape)
        return pl.BlockSpec(shape, lambda i, _n=nd: (0,) * _n)

    in_specs = [
        pl.BlockSpec((BM, N, ATOM_FDIM), lambda i: (i, 0, 0)),
        pl.BlockSpec((BM, N, BOND_FDIM), lambda i: (i, 0, 0)),
        pl.BlockSpec((BM, N * NBS), lambda i: (i, 0)),
        pl.BlockSpec((BM, N * NBS), lambda i: (i, 0)),
        pl.BlockSpec((BM, N * NBS), lambda i: (i, 0)),
        pl.BlockSpec((BM, N), lambda i: (i, 0)),
        rep((1, N, N * NBS)),
    ] + [rep(a.shape) for a in flat]

    out_shape = (
        jax.ShapeDtypeStruct((B, N, H), jnp.float32),
        jax.ShapeDtypeStruct((B, 1, H), jnp.float32),
    )
    out_specs = (
        pl.BlockSpec((BM, N, H), lambda i: (i, 0, 0)),
        pl.BlockSpec((BM, 1, H), lambda i: (i, 0, 0)),
    )
    vf, sf = pl.pallas_call(
        _body,
        grid=(B // BM,),
        in_specs=in_specs,
        out_specs=out_specs,
        out_shape=out_shape,
    )(atom_fea, bond_fea, anb2, bnb2, nmask2, dmask2,
      jnp.asarray(_PAT), *flat)
    return vf, sf


# speedup vs baseline: 1.8406x; 1.1406x over previous
import jax
import jax.numpy as jnp
import numpy as np
from jax.experimental import pallas as pl

ATOM_FDIM = 128
BOND_FDIM = 6
H = 64
KHEAD = 2
DEPTH = 2
N = 64
NBS = 8
BM = 32

_F32 = jnp.float32

_PAT = np.repeat(np.eye(N, dtype=np.float32), NBS, axis=1).reshape(1, N, N * NBS)


def _lrelu(x, s):
    return jnp.maximum(x, s * x)


def _bdot(a, b):
    return jax.lax.dot_general(
        a, b, (((2,), (1,)), ((0,), (0,))), preferred_element_type=_F32)


def _dot(x, w, b=None):
    r = jnp.dot(x, w, preferred_element_type=_F32)
    return r if b is None else r + b


_NSQ = 12
_NB = 15


def _prep_params(params):
    sq, wide, u2s, biases, bmmw = [], [], [], [], []
    biases.append(params["vertex_embedding"][1])
    for it in range(DEPTH):
        sq.extend([
            params["W_a_main"][it][0][0], params["W_main"][it][0][0],
            params["W_a_main"][it][1][0], params["W_main"][it][1][0],
            params["W_a_super"][it][0][0], params["W_a_super"][it][1][0],
            params["W_super_to_main"][it][0], params["W_super"][it][0],
            params["W_zm1"][it][0], params["W_zm2"][it][0],
            params["W_zs1"][it][0], params["W_zs2"][it][0],
        ])
        wide.extend([params["label_U1"][it][0], params["W_main_to_super"][it][0]])
        u2s.append(params["label_U2"][it][0])
        biases.extend([
            params["W_a_main"][it][0][1], params["W_main"][it][0][1],
            params["W_a_main"][it][1][1], params["W_main"][it][1][1],
            params["W_a_super"][it][0][1], params["W_a_super"][it][1][1],
            params["W_super_to_main"][it][1], params["W_super"][it][1],
            params["label_U2"][it][1] * 0.5,
            params["label_U1"][it][1], params["W_main_to_super"][it][1],
            params["W_zm1"][it][1], params["W_zm2"][it][1],
            params["W_zs1"][it][1], params["W_zs2"][it][1],
        ])
        for k in range(KHEAD):
            bmmw.append(params["W_bmm"][it][k][0].reshape(H))
    return [
        params["vertex_embedding"][0].T,
        jnp.stack(sq).transpose(0, 2, 1),
        jnp.stack(wide).transpose(0, 2, 1),
        jnp.stack(u2s).transpose(0, 2, 1),
        jnp.stack(biases),
        jnp.stack(bmmw),
    ]


def _body(af_ref, bf_ref, anb_ref, bnb_ref, nmask_ref, dmask_ref, pat_ref,
          vew_ref, sq_ref, wide_ref, u2_ref, bias_ref, bmmw_ref,
          vf_out, sf_out):
    SQ = sq_ref[...]
    WIDE = wide_ref[...]
    U2T = u2_ref[...]
    BIAS = bias_ref[...]

    AF = af_ref[...].reshape(BM * N, ATOM_FDIM)
    bf = bf_ref[...].reshape(BM * N, BOND_FDIM)
    anb3 = anb_ref[...].reshape(BM, 1, N * NBS)
    bnb3 = bnb_ref[...].reshape(BM, 1, N * NBS)
    nmask_row = nmask_ref[...].reshape(BM, 1, N * NBS)
    dmask_row = dmask_ref[...]
    dmask3 = dmask_row.reshape(BM, 1, N)

    vf = _lrelu(_dot(AF, vew_ref[...], BIAS[0:1]), 0.01)
    sf = _bdot(dmask3, vf.reshape(BM, N, H)).reshape(BM, H)

    iota64 = jax.lax.broadcasted_iota(jnp.int16, (BM, N, N * NBS), 1)
    ohT = jnp.concatenate(
        [anb3 == iota64, bnb3 == iota64], axis=1).astype(jnp.bfloat16)

    R = (pat_ref[...] * nmask_row).astype(jnp.bfloat16)

    for it in range(DEPTH):
        o, ob, ow = it * _NSQ, 1 + it * _NB, 2 * it
        sfW = jnp.concatenate([SQ[o + 4], SQ[o + 5], SQ[o + 6], SQ[o + 7]],
                              axis=1)
        sfb = jnp.concatenate(
            [BIAS[ob + 4:ob + 5], BIAS[ob + 5:ob + 6], BIAS[ob + 6:ob + 7],
             BIAS[ob + 7:ob + 8]], axis=1)
        T = jnp.tanh(_dot(sf, sfW, sfb))
        BMW = bmmw_ref[...]
        bmmw2 = jnp.concatenate([BMW[2 * it:2 * it + 1],
                                 BMW[2 * it + 1:2 * it + 2]], axis=1)
        Cpair = (T[:, :2 * H] * bmmw2).reshape(BM, 2 * H)
        s2m = T[:, 2 * H:3 * H]
        ss = T[:, 3 * H:]
        heads = []
        for k in range(KHEAD):
            AM = jnp.tanh(_dot(vf, SQ[o + 2 * k], BIAS[ob + 2 * k:ob + 2 * k + 1]))
            V = _dot(vf, SQ[o + 2 * k + 1], BIAS[ob + 2 * k + 1:ob + 2 * k + 2])
            Ck = Cpair[:, k * H:(k + 1) * H].reshape(BM, 1, H)
            a = jax.lax.dot_general(
                Ck, AM.reshape(BM, N, H), (((2,), (2,)), ((0,), (0,))),
                preferred_element_type=_F32)
            aexp = jnp.exp(a) * dmask3
            den = jnp.sum(aexp, axis=-1, keepdims=True)
            pooled = _bdot(aexp, V.reshape(BM, N, H))
            heads.append(pooled / (den + 1e-6))
        h0 = heads[0].reshape(BM, H)
        h1 = heads[1].reshape(BM, H)
        m2sT = WIDE[ow + 1]
        mts = jnp.tanh(_dot(h0, m2sT[:H]) + _dot(h1, m2sT[H:])
                       + BIAS[ob + 10:ob + 11])

        u2b2 = BIAS[ob + 8:ob + 9]
        Pv = _dot(vf, U2T[it][:H], u2b2)
        Pe = _dot(bf, U2T[it][H:], u2b2)
        Pcat = jnp.concatenate(
            [Pv.reshape(BM, N, H), Pe.reshape(BM, N, H)], axis=1)
        G = jax.lax.dot_general(
            ohT, Pcat, (((1,), (1,)), ((0,), (0,))),
            preferred_element_type=_F32)
        L = _lrelu(G, 0.1)
        nei = _bdot(R, L).reshape(BM * N, H)
        U1T = WIDE[ow]
        main_self = _lrelu(
            _dot(vf, U1T[:H]) + _dot(nei, U1T[H:]) + BIAS[ob + 9:ob + 10], 0.1)

        zm_in = _dot(main_self, SQ[o + 8], BIAS[ob + 11:ob + 12]
                     ).reshape(BM, N, H)
        zm_sup = _dot(s2m, SQ[o + 9], BIAS[ob + 12:ob + 13])
        zm = jax.nn.sigmoid(zm_in + zm_sup.reshape(BM, 1, H))
        vf = ((1 - zm) * main_self.reshape(BM, N, H)
              + zm * s2m.reshape(BM, 1, H)).reshape(BM * N, H)
        zs = jax.nn.sigmoid(_dot(ss, SQ[o + 10], BIAS[ob + 13:ob + 14])
                            + _dot(mts, SQ[o + 11], BIAS[ob + 14:ob + 15]))
        sf = (1 - zs) * ss + zs * mts

    vf_out[...] = vf.reshape(BM, N, H)
    sf_out[...] = sf.reshape(BM, 1, H)


def kernel(batch_size, atom_fea, bond_fea, d_anb, d_bnb, d_nbs_mask, d_mask, params):
    B = atom_fea.shape[0]
    flat = _prep_params(params)

    anb2 = d_anb.reshape(B, N * NBS).astype(jnp.int16)
    bnb2 = d_bnb.reshape(B, N * NBS).astype(jnp.int16)
    nmask2 = d_nbs_mask.reshape(B, N * NBS)
    dmask2 = d_mask

    def rep(shape):
        nd = len(shape)
        return pl.BlockSpec(shape, lambda i, _n=nd: (0,) * _n)

    in_specs = [
        pl.BlockSpec((BM, N, ATOM_FDIM), lambda i: (i, 0, 0)),
        pl.BlockSpec((BM, N, BOND_FDIM), lambda i: (i, 0, 0)),
        pl.BlockSpec((BM, N * NBS), lambda i: (i, 0)),
        pl.BlockSpec((BM, N * NBS), lambda i: (i, 0)),
        pl.BlockSpec((BM, N * NBS), lambda i: (i, 0)),
        pl.BlockSpec((BM, N), lambda i: (i, 0)),
        rep((1, N, N * NBS)),
    ] + [rep(a.shape) for a in flat]

    out_shape = (
        jax.ShapeDtypeStruct((B, N, H), jnp.float32),
        jax.ShapeDtypeStruct((B, 1, H), jnp.float32),
    )
    out_specs = (
        pl.BlockSpec((BM, N, H), lambda i: (i, 0, 0)),
        pl.BlockSpec((BM, 1, H), lambda i: (i, 0, 0)),
    )
    vf, sf = pl.pallas_call(
        _body,
        grid=(B // BM,),
        in_specs=in_specs,
        out_specs=out_specs,
        out_shape=out_shape,
    )(atom_fea, bond_fea, anb2, bnb2, nmask2, dmask2,
      jnp.asarray(_PAT), *flat)
    return vf, sf


# speedup vs baseline: 2.0668x; 1.1229x over previous
import jax
import jax.numpy as jnp
import numpy as np
from jax.experimental import pallas as pl

ATOM_FDIM = 128
BOND_FDIM = 6
H = 64
KHEAD = 2
DEPTH = 2
N = 64
NBS = 8
BM = 32

_F32 = jnp.float32

_PAT = np.repeat(np.eye(N, dtype=np.float32), NBS, axis=1).reshape(1, N, N * NBS)


def _lrelu(x, s):
    return jnp.maximum(x, s * x)


def _bdot(a, b):
    return jax.lax.dot_general(
        a, b, (((2,), (1,)), ((0,), (0,))), preferred_element_type=_F32)


def _dot(x, w, b=None):
    r = jnp.dot(x, w, preferred_element_type=_F32)
    return r if b is None else r + b


_NSQ = 12
_NB = 15


def _prep_params(params):
    sq, wide, u2s, biases, bmmw = [], [], [], [], []
    biases.append(params["vertex_embedding"][1])
    for it in range(DEPTH):
        sq.extend([
            params["W_a_main"][it][0][0], params["W_main"][it][0][0],
            params["W_a_main"][it][1][0], params["W_main"][it][1][0],
            params["W_a_super"][it][0][0], params["W_a_super"][it][1][0],
            params["W_super_to_main"][it][0], params["W_super"][it][0],
            params["W_zm1"][it][0], params["W_zm2"][it][0],
            params["W_zs1"][it][0], params["W_zs2"][it][0],
        ])
        wide.extend([params["label_U1"][it][0], params["W_main_to_super"][it][0]])
        u2s.append(params["label_U2"][it][0])
        biases.extend([
            params["W_a_main"][it][0][1], params["W_main"][it][0][1],
            params["W_a_main"][it][1][1], params["W_main"][it][1][1],
            params["W_a_super"][it][0][1], params["W_a_super"][it][1][1],
            params["W_super_to_main"][it][1], params["W_super"][it][1],
            params["label_U2"][it][1] * 0.5,
            params["label_U1"][it][1], params["W_main_to_super"][it][1],
            params["W_zm1"][it][1], params["W_zm2"][it][1],
            params["W_zs1"][it][1], params["W_zs2"][it][1],
        ])
        for k in range(KHEAD):
            bmmw.append(params["W_bmm"][it][k][0].reshape(H))
    return [
        params["vertex_embedding"][0].T,
        jnp.stack(sq).transpose(0, 2, 1),
        jnp.stack(wide).transpose(0, 2, 1),
        jnp.stack(u2s).transpose(0, 2, 1),
        jnp.stack(biases),
        jnp.stack(bmmw),
    ]


def _body(af_ref, bf_ref, anb_ref, bnb_ref, nmask_ref, dmask_ref, pat_ref,
          vew_ref, sq_ref, wide_ref, u2_ref, bias_ref, bmmw_ref,
          vf_out, sf_out):
    SQ = sq_ref[...]
    WIDE = wide_ref[...]
    U2T = u2_ref[...]
    BIAS = bias_ref[...]

    AF = af_ref[...].reshape(BM * N, ATOM_FDIM)
    bf = bf_ref[...].reshape(BM * N, BOND_FDIM)
    anb3 = anb_ref[...].reshape(BM, 1, N * NBS)
    bnb3 = bnb_ref[...].reshape(BM, 1, N * NBS)
    nmask_row = nmask_ref[...].reshape(BM, 1, N * NBS)
    dmask_row = dmask_ref[...]
    dmask3 = dmask_row.reshape(BM, 1, N)

    vf = _lrelu(_dot(AF, vew_ref[...], BIAS[0:1]), 0.01)
    sf = _bdot(dmask3, vf.reshape(BM, N, H)).reshape(BM, H)

    iota64 = jax.lax.broadcasted_iota(jnp.int16, (BM, N, N * NBS), 1)
    ohT = jnp.concatenate(
        [anb3 == iota64, bnb3 == iota64], axis=1).astype(jnp.bfloat16)

    R = (pat_ref[...] * nmask_row).astype(jnp.bfloat16)

    for it in range(DEPTH):
        o, ob, ow = it * _NSQ, 1 + it * _NB, 2 * it
        sfW = jnp.concatenate([SQ[o + 4], SQ[o + 5], SQ[o + 6], SQ[o + 7]],
                              axis=1)
        sfb = jnp.concatenate(
            [BIAS[ob + 4:ob + 5], BIAS[ob + 5:ob + 6], BIAS[ob + 6:ob + 7],
             BIAS[ob + 7:ob + 8]], axis=1)
        T = jnp.tanh(_dot(sf, sfW, sfb))
        BMW = bmmw_ref[...]
        bmmw2 = jnp.concatenate([BMW[2 * it:2 * it + 1],
                                 BMW[2 * it + 1:2 * it + 2]], axis=1)
        Cpair = (T[:, :2 * H] * bmmw2).reshape(BM, 2 * H)
        s2m = T[:, 2 * H:3 * H]
        ss = T[:, 3 * H:]
        heads = []
        for k in range(KHEAD):
            AM = jnp.tanh(_dot(vf, SQ[o + 2 * k], BIAS[ob + 2 * k:ob + 2 * k + 1]))
            V = _dot(vf, SQ[o + 2 * k + 1], BIAS[ob + 2 * k + 1:ob + 2 * k + 2])
            Ck = Cpair[:, k * H:(k + 1) * H].reshape(BM, 1, H)
            a = jax.lax.dot_general(
                Ck, AM.reshape(BM, N, H), (((2,), (2,)), ((0,), (0,))),
                preferred_element_type=_F32)
            aexp = jnp.exp(a) * dmask3
            den = jnp.sum(aexp, axis=-1, keepdims=True)
            pooled = _bdot(aexp, V.reshape(BM, N, H))
            heads.append(pooled / (den + 1e-6))
        h0 = heads[0].reshape(BM, H)
        h1 = heads[1].reshape(BM, H)
        m2sT = WIDE[ow + 1]
        mts = jnp.tanh(_dot(h0, m2sT[:H]) + _dot(h1, m2sT[H:])
                       + BIAS[ob + 10:ob + 11])

        u2b2 = BIAS[ob + 8:ob + 9]
        Pv = _dot(vf, U2T[it][:H], u2b2)
        Pe = _dot(bf, U2T[it][H:], u2b2)
        Pcat = jnp.concatenate(
            [Pv.reshape(BM, N, H), Pe.reshape(BM, N, H)],
            axis=1).astype(jnp.bfloat16)
        G = jax.lax.dot_general(
            ohT, Pcat, (((1,), (1,)), ((0,), (0,))),
            preferred_element_type=_F32).astype(jnp.bfloat16)
        L = _lrelu(G, jnp.bfloat16(0.1))
        nei = _bdot(R, L).reshape(BM * N, H)
        U1T = WIDE[ow]
        main_self = _lrelu(
            _dot(vf, U1T[:H]) + _dot(nei, U1T[H:]) + BIAS[ob + 9:ob + 10], 0.1)

        zm_in = _dot(main_self, SQ[o + 8], BIAS[ob + 11:ob + 12]
                     ).reshape(BM, N, H)
        zm_sup = _dot(s2m, SQ[o + 9], BIAS[ob + 12:ob + 13])
        zm = jax.nn.sigmoid(zm_in + zm_sup.reshape(BM, 1, H))
        vf = ((1 - zm) * main_self.reshape(BM, N, H)
              + zm * s2m.reshape(BM, 1, H)).reshape(BM * N, H)
        zs = jax.nn.sigmoid(_dot(ss, SQ[o + 10], BIAS[ob + 13:ob + 14])
                            + _dot(mts, SQ[o + 11], BIAS[ob + 14:ob + 15]))
        sf = (1 - zs) * ss + zs * mts

    vf_out[...] = vf.reshape(BM, N, H)
    sf_out[...] = sf.reshape(BM, 1, H)


def kernel(batch_size, atom_fea, bond_fea, d_anb, d_bnb, d_nbs_mask, d_mask, params):
    B = atom_fea.shape[0]
    flat = _prep_params(params)

    anb2 = d_anb.reshape(B, N * NBS).astype(jnp.int16)
    bnb2 = d_bnb.reshape(B, N * NBS).astype(jnp.int16)
    nmask2 = d_nbs_mask.reshape(B, N * NBS)
    dmask2 = d_mask

    def rep(shape):
        nd = len(shape)
        return pl.BlockSpec(shape, lambda i, _n=nd: (0,) * _n)

    in_specs = [
        pl.BlockSpec((BM, N, ATOM_FDIM), lambda i: (i, 0, 0)),
        pl.BlockSpec((BM, N, BOND_FDIM), lambda i: (i, 0, 0)),
        pl.BlockSpec((BM, N * NBS), lambda i: (i, 0)),
        pl.BlockSpec((BM, N * NBS), lambda i: (i, 0)),
        pl.BlockSpec((BM, N * NBS), lambda i: (i, 0)),
        pl.BlockSpec((BM, N), lambda i: (i, 0)),
        rep((1, N, N * NBS)),
    ] + [rep(a.shape) for a in flat]

    out_shape = (
        jax.ShapeDtypeStruct((B, N, H), jnp.float32),
        jax.ShapeDtypeStruct((B, 1, H), jnp.float32),
    )
    out_specs = (
        pl.BlockSpec((BM, N, H), lambda i: (i, 0, 0)),
        pl.BlockSpec((BM, 1, H), lambda i: (i, 0, 0)),
    )
    vf, sf = pl.pallas_call(
        _body,
        grid=(B // BM,),
        in_specs=in_specs,
        out_specs=out_specs,
        out_shape=out_shape,
    )(atom_fea, bond_fea, anb2, bnb2, nmask2, dmask2,
      jnp.asarray(_PAT), *flat)
    return vf, sf
